# baseline (speedup 1.0000x reference)
import functools

import jax
import jax.numpy as jnp
from jax import lax
from jax.experimental import pallas as pl
from jax.experimental.pallas import tpu as pltpu
from jax.experimental.pallas import tpu_sc as plsc

B = 4096
S = 100
V = 1000
E = 128
H = 128
O = 1000
VP = 1024

NC = 2
NS = 16
NW = NC * NS
L = 16
RW = B // NW
RC = 64
NCHUNK = RW // RC

BBLK = 2048


def _counts_body(ids_hbm, counts_hbm, ids_v, counts_v):
    wid = lax.axis_index("s") * NC + lax.axis_index("c")
    base = wid * RW
    pltpu.sync_copy(ids_hbm.at[:, pl.ds(base, RW)], ids_v)

    lane = lax.iota(jnp.int32, L)
    ones = jnp.full((L,), 1.0, jnp.float32)
    zeros = jnp.zeros((L,), jnp.float32)

    for chunk in range(NCHUNK):
        ZU = 16
        def zbody(i, _):
            for u in range(ZU):
                counts_v[pl.ds((i * ZU + u) * L, L)] = zeros
            return 0
        lax.fori_loop(0, RC * VP // (L * ZU), zbody, 0)

        row_base = []
        for g in range(RC // L):
            r = jnp.full((L,), g * L, jnp.int32) + lane
            row_base.append((r >> 3) * (8 * VP) + (r & 7) * 128)

        def sbody(s, _):
            for g in range(RC // L):
                ids16 = ids_v[s, pl.ds(chunk * RC + g * L, L)]
                idoff = ids16 + (ids16 >> 7) * (8 * 128 - 128)
                plsc.addupdate_scatter(counts_v, [row_base[g] + idoff], ones)
            return 0
        lax.fori_loop(0, S, sbody, 0)

        pltpu.sync_copy(
            counts_v,
            counts_hbm.at[pl.ds((base + chunk * RC) * VP, RC * VP)])


@functools.cache
def _counts_kernel():
    return pl.kernel(
        _counts_body,
        out_type=jax.ShapeDtypeStruct((B * VP,), jnp.float32),
        mesh=plsc.VectorSubcoreMesh(core_axis_name="c", subcore_axis_name="s"),
        compiler_params=pltpu.CompilerParams(use_tc_tiling_on_sc=False,
                                             needs_layout_passes=False),
        scratch_types=[
            pltpu.VMEM((S, RW), jnp.int32),
            pltpu.VMEM((RC * VP,), jnp.float32),
        ],
    )


def _mlp_body(counts_ref, emb_ref, w1t_ref, b1t_ref, w2t_ref, b2t_ref, out_ref):
    pooled = jnp.zeros((BBLK, E), jnp.float32)
    for tc in range(VP // 128):
        c = counts_ref[:, tc, :, :].reshape(BBLK, 128)
        e = emb_ref[pl.ds(tc * 128, 128), :]
        pooled = pooled + jax.lax.dot(
            c, e, precision=jax.lax.Precision.DEFAULT,
            preferred_element_type=jnp.float32)
    pooled = pooled * (1.0 / S)
    pooled_t = pooled.T
    h_t = jnp.maximum(
        jax.lax.dot(w1t_ref[...], pooled_t,
                    precision=jax.lax.Precision.DEFAULT,
                    preferred_element_type=jnp.float32) + b1t_ref[...], 0.0)
    logits_t = jax.lax.dot(
        w2t_ref[...], h_t,
        precision=jax.lax.Precision.DEFAULT,
        preferred_element_type=jnp.float32) + b2t_ref[...]
    m = jnp.max(logits_t, axis=0, keepdims=True)
    e = jnp.exp(logits_t - m)
    out_ref[...] = e / jnp.sum(e, axis=0, keepdims=True)


def _mlp_call(counts, emb_pad, W1t, b1t, W2t, b2t):
    grid = (B // BBLK,)
    return pl.pallas_call(
        _mlp_body,
        grid=grid,
        in_specs=[
            pl.BlockSpec((BBLK // 8, VP // 128, 8, 128), lambda i: (i, 0, 0, 0)),
            pl.BlockSpec((VP, E), lambda i: (0, 0)),
            pl.BlockSpec((H, E), lambda i: (0, 0)),
            pl.BlockSpec((H, 1), lambda i: (0, 0)),
            pl.BlockSpec((O, H), lambda i: (0, 0)),
            pl.BlockSpec((O, 1), lambda i: (0, 0)),
        ],
        out_specs=pl.BlockSpec((O, BBLK), lambda i: (0, i)),
        out_shape=jax.ShapeDtypeStruct((O, B), jnp.float32),
    )(counts, emb_pad, W1t, b1t, W2t, b2t)


def kernel(input_ids, emb_table, W1, b1, W2, b2):
    ids_t = input_ids.astype(jnp.int32).T
    counts = _counts_kernel()(ids_t).reshape(B // 8, VP // 128, 8, 128)
    emb_pad = jnp.pad(emb_table, ((0, VP - V), (0, 0)))
    out_t = _mlp_call(counts, emb_pad, W1.T, b1.reshape(H, 1),
                      W2.T, b2.reshape(O, 1))
    return out_t.T


# speedup vs baseline: 1.0807x; 1.0807x over previous
import functools

import jax
import jax.numpy as jnp
from jax import lax
from jax.experimental import pallas as pl
from jax.experimental.pallas import tpu as pltpu
from jax.experimental.pallas import tpu_sc as plsc

B = 4096
S = 100
V = 1000
E = 128
H = 128
O = 1000
VP = 1024

NC = 2
NS = 16
NW = NC * NS
L = 16
RW = B // NW
RC = 64
NCHUNK = RW // RC

BBLK = 1024


RB = 16
NB = RW // RB


def _counts_body(ids_hbm, counts_hbm, ids_v, buf0, buf1, sem0, sem1):
    wid = lax.axis_index("s") * NC + lax.axis_index("c")
    base = wid * RW
    pltpu.sync_copy(ids_hbm.at[:, pl.ds(base, RW)], ids_v)

    lane = lax.iota(jnp.int32, L)
    ones = jnp.full((L,), 1.0, jnp.float32)
    zeros = jnp.zeros((L,), jnp.float32)
    row_base = (lane >> 3) * (8 * VP) + (lane & 7) * 128

    bufs = (buf0, buf1)
    sems = (sem0, sem1)

    def out_slice(rb):
        return counts_hbm.at[pl.ds((base + rb * RB) * VP, RB * VP)]

    for rb in range(NB):
        buf, sem = bufs[rb % 2], sems[rb % 2]
        if rb >= 2:
            pltpu.make_async_copy(buf, out_slice(rb - 2), sem).wait()

        ZU = 16
        def zbody(i, _):
            for u in range(ZU):
                buf[pl.ds((i * ZU + u) * L, L)] = zeros
            return 0
        lax.fori_loop(0, RB * VP // (L * ZU), zbody, 0)

        def sbody(s, _):
            ids16 = ids_v[s, pl.ds(rb * RB, L)]
            idoff = ids16 + (ids16 >> 7) * (8 * 128 - 128)
            plsc.addupdate_scatter(buf, [row_base + idoff], ones)
            return 0
        lax.fori_loop(0, S, sbody, 0)

        pltpu.make_async_copy(buf, out_slice(rb), sem).start()

    pltpu.make_async_copy(bufs[0], out_slice(NB - 2), sems[0]).wait()
    pltpu.make_async_copy(bufs[1], out_slice(NB - 1), sems[1]).wait()


@functools.cache
def _counts_kernel():
    return pl.kernel(
        _counts_body,
        out_type=jax.ShapeDtypeStruct((B * VP,), jnp.float32),
        mesh=plsc.VectorSubcoreMesh(core_axis_name="c", subcore_axis_name="s"),
        compiler_params=pltpu.CompilerParams(use_tc_tiling_on_sc=False,
                                             needs_layout_passes=False),
        scratch_types=[
            pltpu.VMEM((S, RW), jnp.int32),
            pltpu.VMEM((RB * VP,), jnp.float32),
            pltpu.VMEM((RB * VP,), jnp.float32),
            pltpu.SemaphoreType.DMA,
            pltpu.SemaphoreType.DMA,
        ],
    )


def _mlp_body(counts_ref, emb_ref, w1t_ref, b1t_ref, w2t_ref, b2t_ref, out_ref):
    pooled = jnp.zeros((BBLK, E), jnp.float32)
    for tc in range(VP // 128):
        c = counts_ref[:, tc, :, :].reshape(BBLK, 128)
        e = emb_ref[pl.ds(tc * 128, 128), :]
        pooled = pooled + jax.lax.dot(
            c, e, precision=jax.lax.Precision.DEFAULT,
            preferred_element_type=jnp.float32)
    pooled = pooled * (1.0 / S)
    pooled_t = pooled.T
    h_t = jnp.maximum(
        jax.lax.dot(w1t_ref[...], pooled_t,
                    precision=jax.lax.Precision.DEFAULT,
                    preferred_element_type=jnp.float32) + b1t_ref[...], 0.0)
    logits_t = jax.lax.dot(
        w2t_ref[...], h_t,
        precision=jax.lax.Precision.DEFAULT,
        preferred_element_type=jnp.float32) + b2t_ref[...]
    m = jnp.max(logits_t, axis=0, keepdims=True)
    e = jnp.exp(logits_t - m)
    out_ref[...] = e / jnp.sum(e, axis=0, keepdims=True)


def _mlp_call(counts, emb_pad, W1t, b1t, W2t, b2t):
    grid = (B // BBLK,)
    return pl.pallas_call(
        _mlp_body,
        grid=grid,
        in_specs=[
            pl.BlockSpec((BBLK // 8, VP // 128, 8, 128), lambda i: (i, 0, 0, 0)),
            pl.BlockSpec((VP, E), lambda i: (0, 0)),
            pl.BlockSpec((H, E), lambda i: (0, 0)),
            pl.BlockSpec((H, 1), lambda i: (0, 0)),
            pl.BlockSpec((O, H), lambda i: (0, 0)),
            pl.BlockSpec((O, 1), lambda i: (0, 0)),
        ],
        out_specs=pl.BlockSpec((O, BBLK), lambda i: (0, i)),
        out_shape=jax.ShapeDtypeStruct((O, B), jnp.float32),
    )(counts, emb_pad, W1t, b1t, W2t, b2t)


def kernel(input_ids, emb_table, W1, b1, W2, b2):
    ids_t = input_ids.astype(jnp.int32).T
    counts = _counts_kernel()(ids_t).reshape(B // 8, VP // 128, 8, 128)
    emb_pad = jnp.pad(emb_table, ((0, VP - V), (0, 0)))
    out_t = _mlp_call(counts, emb_pad, W1.T, b1.reshape(H, 1),
                      W2.T, b2.reshape(O, 1))
    return out_t.T


# speedup vs baseline: 1.1012x; 1.0190x over previous
import functools

import jax
import jax.numpy as jnp
from jax import lax
from jax.experimental import pallas as pl
from jax.experimental.pallas import tpu as pltpu
from jax.experimental.pallas import tpu_sc as plsc

B = 4096
S = 100
V = 1000
E = 128
H = 128
O = 1000
VP = 1024

NC = 2
NS = 16
NW = NC * NS
L = 16
RW = B // NW
RC = 64
NCHUNK = RW // RC

BBLK = 1024


BH = B // 2
RWH = BH // NW
RB = 16
NB = RWH // RB


def _make_counts_body(half):
    def _counts_body(ids_hbm, counts_hbm, ids_v, buf0, buf1, sem0, sem1):
        wid = lax.axis_index("s") * NC + lax.axis_index("c")
        base = wid * RWH
        pltpu.sync_copy(
            ids_hbm.at[:, pl.ds(half * BH + base, RWH)], ids_v)

        lane = lax.iota(jnp.int32, L)
        ones = jnp.full((L,), 1.0, jnp.float32)
        zeros = jnp.zeros((L,), jnp.float32)
        row_base = (lane >> 3) * (8 * VP) + (lane & 7) * 128

        bufs = (buf0, buf1)
        sems = (sem0, sem1)

        def out_slice(rb):
            return counts_hbm.at[pl.ds((base + rb * RB) * VP, RB * VP)]

        for rb in range(NB):
            buf, sem = bufs[rb % 2], sems[rb % 2]
            if rb >= 2:
                pltpu.make_async_copy(buf, out_slice(rb - 2), sem).wait()

            ZU = 16
            def zbody(i, _):
                for u in range(ZU):
                    buf[pl.ds((i * ZU + u) * L, L)] = zeros
                return 0
            lax.fori_loop(0, RB * VP // (L * ZU), zbody, 0)

            def sbody(s, _):
                ids16 = ids_v[s, pl.ds(rb * RB, L)]
                idoff = ids16 + (ids16 >> 7) * (8 * 128 - 128)
                plsc.addupdate_scatter(buf, [row_base + idoff], ones)
                return 0
            lax.fori_loop(0, S, sbody, 0)

            pltpu.make_async_copy(buf, out_slice(rb), sem).start()

        pltpu.make_async_copy(bufs[0], out_slice(NB - 2), sems[0]).wait()
        pltpu.make_async_copy(bufs[1], out_slice(NB - 1), sems[1]).wait()

    return _counts_body


@functools.cache
def _counts_kernel(half):
    return pl.kernel(
        _make_counts_body(half),
        out_type=jax.ShapeDtypeStruct((BH * VP,), jnp.float32),
        mesh=plsc.VectorSubcoreMesh(core_axis_name="c", subcore_axis_name="s"),
        compiler_params=pltpu.CompilerParams(use_tc_tiling_on_sc=False,
                                             needs_layout_passes=False),
        scratch_types=[
            pltpu.VMEM((S, RWH), jnp.int32),
            pltpu.VMEM((RB * VP,), jnp.float32),
            pltpu.VMEM((RB * VP,), jnp.float32),
            pltpu.SemaphoreType.DMA,
            pltpu.SemaphoreType.DMA,
        ],
    )


def _mlp_body(counts_ref, emb_ref, w1t_ref, b1t_ref, w2t_ref, b2t_ref, out_ref):
    pooled = jnp.zeros((BBLK, E), jnp.float32)
    for tc in range(VP // 128):
        c = counts_ref[:, tc, :, :].reshape(BBLK, 128)
        e = emb_ref[pl.ds(tc * 128, 128), :]
        pooled = pooled + jax.lax.dot(
            c, e, precision=jax.lax.Precision.DEFAULT,
            preferred_element_type=jnp.float32)
    pooled = pooled * (1.0 / S)
    pooled_t = pooled.T
    h_t = jnp.maximum(
        jax.lax.dot(w1t_ref[...], pooled_t,
                    precision=jax.lax.Precision.DEFAULT,
                    preferred_element_type=jnp.float32) + b1t_ref[...], 0.0)
    logits_t = jax.lax.dot(
        w2t_ref[...], h_t,
        precision=jax.lax.Precision.DEFAULT,
        preferred_element_type=jnp.float32) + b2t_ref[...]
    m = jnp.max(logits_t, axis=0, keepdims=True)
    e = jnp.exp(logits_t - m)
    out_ref[...] = e / jnp.sum(e, axis=0, keepdims=True)


def _mlp_body_alias(counts_ref, emb_ref, w1t_ref, b1t_ref, w2t_ref, b2t_ref,
                    prev_ref, out_ref):
    del prev_ref
    _mlp_body(counts_ref, emb_ref, w1t_ref, b1t_ref, w2t_ref, b2t_ref,
              out_ref)


def _mlp_call_half(half, counts, emb_pad, W1t, b1t, W2t, b2t, prev=None):
    grid = (BH // BBLK,)
    in_specs = [
        pl.BlockSpec((BBLK // 8, VP // 128, 8, 128), lambda i: (i, 0, 0, 0)),
        pl.BlockSpec((VP, E), lambda i: (0, 0)),
        pl.BlockSpec((H, E), lambda i: (0, 0)),
        pl.BlockSpec((H, 1), lambda i: (0, 0)),
        pl.BlockSpec((O, H), lambda i: (0, 0)),
        pl.BlockSpec((O, 1), lambda i: (0, 0)),
    ]
    args = (counts, emb_pad, W1t, b1t, W2t, b2t)
    body = _mlp_body
    kwargs = {}
    if prev is not None:
        in_specs.append(pl.BlockSpec(memory_space=pl.ANY))
        args = args + (prev,)
        body = _mlp_body_alias
        kwargs["input_output_aliases"] = {6: 0}
    off = half * (BH // BBLK)
    return pl.pallas_call(
        body,
        grid=grid,
        in_specs=in_specs,
        out_specs=pl.BlockSpec((O, BBLK), lambda i: (0, off + i)),
        out_shape=jax.ShapeDtypeStruct((O, B), jnp.float32),
        **kwargs,
    )(*args)


def kernel(input_ids, emb_table, W1, b1, W2, b2):
    ids_t = input_ids.astype(jnp.int32).T
    emb_pad = jnp.pad(emb_table, ((0, VP - V), (0, 0)))
    W1t, b1t = W1.T, b1.reshape(H, 1)
    W2t, b2t = W2.T, b2.reshape(O, 1)
    c0 = _counts_kernel(0)(ids_t).reshape(BH // 8, VP // 128, 8, 128)
    c1 = _counts_kernel(1)(ids_t).reshape(BH // 8, VP // 128, 8, 128)
    out0 = _mlp_call_half(0, c0, emb_pad, W1t, b1t, W2t, b2t)
    out_t = _mlp_call_half(1, c1, emb_pad, W1t, b1t, W2t, b2t, prev=out0)
    return out_t.T


# speedup vs baseline: 1.1018x; 1.0006x over previous
import functools

import jax
import jax.numpy as jnp
from jax import lax
from jax.experimental import pallas as pl
from jax.experimental.pallas import tpu as pltpu
from jax.experimental.pallas import tpu_sc as plsc

B = 4096
S = 100
V = 1000
E = 128
H = 128
O = 1000
VP = 1024

NC = 2
NS = 16
NW = NC * NS
L = 16
RW = B // NW
RC = 64
NCHUNK = RW // RC

BBLK = 1024


BH = B // 2
RWH = BH // NW
RB = 16
NB = RWH // RB


def _make_counts_body(half):
    def _counts_body(ids_hbm, counts_hbm, ids_v, buf0, buf1, sem0, sem1):
        wid = lax.axis_index("s") * NC + lax.axis_index("c")
        base = wid * RWH
        pltpu.sync_copy(ids_hbm.at[:, pl.ds(base, RWH)], ids_v)

        lane = lax.iota(jnp.int32, L)
        ones = jnp.full((L,), 1.0, jnp.float32)
        zeros = jnp.zeros((L,), jnp.float32)
        row_base = (lane >> 3) * (8 * VP) + (lane & 7) * 128

        bufs = (buf0, buf1)
        sems = (sem0, sem1)

        def out_slice(rb):
            return counts_hbm.at[pl.ds((base + rb * RB) * VP, RB * VP)]

        for rb in range(NB):
            buf, sem = bufs[rb % 2], sems[rb % 2]
            if rb >= 2:
                pltpu.make_async_copy(buf, out_slice(rb - 2), sem).wait()

            ZU = 16
            def zbody(i, _):
                for u in range(ZU):
                    buf[pl.ds((i * ZU + u) * L, L)] = zeros
                return 0
            lax.fori_loop(0, RB * VP // (L * ZU), zbody, 0)

            def sbody(s, _):
                ids16 = ids_v[s, pl.ds(rb * RB, L)]
                idoff = ids16 + (ids16 >> 7) * (8 * 128 - 128)
                plsc.addupdate_scatter(buf, [row_base + idoff], ones)
                return 0
            lax.fori_loop(0, S, sbody, 0)

            pltpu.make_async_copy(buf, out_slice(rb), sem).start()

        pltpu.make_async_copy(bufs[0], out_slice(NB - 2), sems[0]).wait()
        pltpu.make_async_copy(bufs[1], out_slice(NB - 1), sems[1]).wait()

    return _counts_body


@functools.cache
def _counts_kernel(half):
    return pl.kernel(
        _make_counts_body(half),
        out_type=jax.ShapeDtypeStruct((BH * VP,), jnp.float32),
        mesh=plsc.VectorSubcoreMesh(core_axis_name="c", subcore_axis_name="s"),
        compiler_params=pltpu.CompilerParams(use_tc_tiling_on_sc=False,
                                             needs_layout_passes=False),
        scratch_types=[
            pltpu.VMEM((S, RWH), jnp.int32),
            pltpu.VMEM((RB * VP,), jnp.float32),
            pltpu.VMEM((RB * VP,), jnp.float32),
            pltpu.SemaphoreType.DMA,
            pltpu.SemaphoreType.DMA,
        ],
    )


def _mlp_body(counts_ref, emb_ref, w1t_ref, b1t_ref, w2t_ref, b2t_ref, out_ref):
    pooled = jnp.zeros((BBLK, E), jnp.float32)
    for tc in range(VP // 128):
        c = counts_ref[:, tc, :, :].reshape(BBLK, 128)
        e = emb_ref[pl.ds(tc * 128, 128), :]
        pooled = pooled + jax.lax.dot(
            c, e, precision=jax.lax.Precision.DEFAULT,
            preferred_element_type=jnp.float32)
    pooled = pooled * (1.0 / S)
    pooled_t = pooled.T
    h_t = jnp.maximum(
        jax.lax.dot(w1t_ref[...], pooled_t,
                    precision=jax.lax.Precision.DEFAULT,
                    preferred_element_type=jnp.float32) + b1t_ref[...], 0.0)
    logits_t = jax.lax.dot(
        w2t_ref[...], h_t,
        precision=jax.lax.Precision.DEFAULT,
        preferred_element_type=jnp.float32) + b2t_ref[...]
    m = jnp.max(logits_t, axis=0, keepdims=True)
    e = jnp.exp(logits_t - m)
    out_ref[...] = e / jnp.sum(e, axis=0, keepdims=True)


def _mlp_body_alias(counts_ref, emb_ref, w1t_ref, b1t_ref, w2t_ref, b2t_ref,
                    prev_ref, out_ref):
    del prev_ref
    _mlp_body(counts_ref, emb_ref, w1t_ref, b1t_ref, w2t_ref, b2t_ref,
              out_ref)


def _mlp_call_half(half, counts, emb_pad, W1t, b1t, W2t, b2t, prev=None):
    grid = (BH // BBLK,)
    in_specs = [
        pl.BlockSpec((BBLK // 8, VP // 128, 8, 128), lambda i: (i, 0, 0, 0)),
        pl.BlockSpec((VP, E), lambda i: (0, 0)),
        pl.BlockSpec((H, E), lambda i: (0, 0)),
        pl.BlockSpec((H, 1), lambda i: (0, 0)),
        pl.BlockSpec((O, H), lambda i: (0, 0)),
        pl.BlockSpec((O, 1), lambda i: (0, 0)),
    ]
    args = (counts, emb_pad, W1t, b1t, W2t, b2t)
    body = _mlp_body
    kwargs = {}
    if prev is not None:
        in_specs.append(pl.BlockSpec(memory_space=pl.ANY))
        args = args + (prev,)
        body = _mlp_body_alias
        kwargs["input_output_aliases"] = {6: 0}
    off = half * (BH // BBLK)
    return pl.pallas_call(
        body,
        grid=grid,
        in_specs=in_specs,
        out_specs=pl.BlockSpec((O, BBLK), lambda i: (0, off + i)),
        out_shape=jax.ShapeDtypeStruct((O, B), jnp.float32),
        **kwargs,
    )(*args)


def kernel(input_ids, emb_table, W1, b1, W2, b2):
    ids_t = input_ids.astype(jnp.int32).T
    emb_pad = jnp.pad(emb_table, ((0, VP - V), (0, 0)))
    W1t, b1t = W1.T, b1.reshape(H, 1)
    W2t, b2t = W2.T, b2.reshape(O, 1)
    c0 = _counts_kernel(0)(ids_t[:, :BH]).reshape(BH // 8, VP // 128, 8, 128)
    c1 = _counts_kernel(1)(ids_t[:, BH:]).reshape(BH // 8, VP // 128, 8, 128)
    out0 = _mlp_call_half(0, c0, emb_pad, W1t, b1t, W2t, b2t)
    out_t = _mlp_call_half(1, c1, emb_pad, W1t, b1t, W2t, b2t, prev=out0)
    return out_t.T
